# Initial kernel scaffold; baseline (speedup 1.0000x reference)
import jax
import jax.numpy as jnp
from jax.experimental import pallas as pl


def kernel(x, edge_index, W1, att_src1, att_dst1, b1, W2, att_src2, att_dst2, b2):
    raise NotImplementedError("write your pallas kernel here")


# speedup vs baseline: 6.7082x; 6.7082x over previous
import functools
import jax
import jax.numpy as jnp
from jax.experimental import pallas as pl
from jax.experimental.pallas import tpu as pltpu

N_NODES = 10000
IN_FEAT = 128
H = 8
HD1 = 8
HD2 = 128
D1 = H * HD1
NEG = 0.2


def _headsum(rows):
    per = rows // H
    r = jax.lax.broadcasted_iota(jnp.int32, (rows, H), 0)
    hh = jax.lax.broadcasted_iota(jnp.int32, (rows, H), 1)
    return jnp.where(r // per == hh, 1.0, 0.0).astype(jnp.float32)


def _headexpand(cols):
    per = cols // H
    hh = jax.lax.broadcasted_iota(jnp.int32, (H, cols), 0)
    c = jax.lax.broadcasted_iota(jnp.int32, (H, cols), 1)
    return jnp.where(c // per == hh, 1.0, 0.0).astype(jnp.float32)


def _dense1_body(x_ref, w1_ref, ats_ref, atd_ref, xp1_ref, asrc_ref, adst_ref, m_ref):
    xp1 = jnp.dot(x_ref[...], w1_ref[...], preferred_element_type=jnp.float32)
    xp1_ref[...] = xp1
    B = _headsum(D1)
    a_s = jnp.dot(xp1 * ats_ref[...], B, preferred_element_type=jnp.float32)
    a_d = jnp.dot(xp1 * atd_ref[...], B, preferred_element_type=jnp.float32)
    asrc_ref[...] = a_s
    adst_ref[...] = a_d
    m = jnp.max(a_s, axis=0, keepdims=True) + jnp.max(a_d, axis=0, keepdims=True)
    m_ref[...] = jnp.where(m >= 0, m, NEG * m)


def _dense1(x, W1, ats_flat, atd_flat):
    return pl.pallas_call(
        _dense1_body,
        out_shape=(
            jax.ShapeDtypeStruct((N_NODES, D1), jnp.float32),
            jax.ShapeDtypeStruct((N_NODES, H), jnp.float32),
            jax.ShapeDtypeStruct((N_NODES, H), jnp.float32),
            jax.ShapeDtypeStruct((1, H), jnp.float32),
        ),
    )(x, W1, ats_flat, atd_flat)


def _dense2_body(uagg_ref, den_ref, b1_ref, w2_ref, ats_ref, atd_ref,
                 h_ref, asrc_ref, adst_ref, m_ref):
    rec = 1.0 / den_ref[...]
    rec64 = jnp.dot(rec, _headexpand(D1), preferred_element_type=jnp.float32)
    hout = uagg_ref[...] * rec64 + b1_ref[...]
    h_ref[...] = hout
    w2 = w2_ref[...]
    B2 = _headsum(H * HD2)
    vs = jnp.dot(w2 * ats_ref[...], B2, preferred_element_type=jnp.float32)
    vd = jnp.dot(w2 * atd_ref[...], B2, preferred_element_type=jnp.float32)
    a_s = jnp.dot(hout, vs, preferred_element_type=jnp.float32)
    a_d = jnp.dot(hout, vd, preferred_element_type=jnp.float32)
    asrc_ref[...] = a_s
    adst_ref[...] = a_d
    m = jnp.max(a_s, axis=0, keepdims=True) + jnp.max(a_d, axis=0, keepdims=True)
    m_ref[...] = jnp.where(m >= 0, m, NEG * m)


def _dense2(uagg1, denom1, b1_flat, W2, ats_flat, atd_flat):
    return pl.pallas_call(
        _dense2_body,
        out_shape=(
            jax.ShapeDtypeStruct((N_NODES, D1), jnp.float32),
            jax.ShapeDtypeStruct((N_NODES, H), jnp.float32),
            jax.ShapeDtypeStruct((N_NODES, H), jnp.float32),
            jax.ShapeDtypeStruct((1, H), jnp.float32),
        ),
    )(uagg1, denom1, b1_flat, W2, ats_flat, atd_flat)


def _final_body(uagg_ref, den_ref, w2_ref, b2_ref, out_ref):
    den = den_ref[...]
    acc = jnp.zeros((uagg_ref.shape[0], HD2), jnp.float32)
    for h in range(H):
        nagg = uagg_ref[:, h * D1:(h + 1) * D1] * (1.0 / den[:, h:h + 1])
        acc = acc + jnp.dot(nagg, w2_ref[:, h * HD2:(h + 1) * HD2],
                            preferred_element_type=jnp.float32)
    out_ref[...] = acc * (1.0 / H) + b2_ref[...]


def _final(uagg2, denom2, W2, b2_flat):
    return pl.pallas_call(
        _final_body,
        out_shape=jax.ShapeDtypeStruct((N_NODES, HD2), jnp.float32),
    )(uagg2, denom2, W2, b2_flat)


def _edge_phase(a_src, a_dst, M, feat, src, dst, per_head_slice):
    alpha = a_src[src] + a_dst[dst]
    alpha = jnp.where(alpha >= 0, alpha, NEG * alpha)
    ex = jnp.exp(alpha - M.reshape(1, H))
    denom = jax.ops.segment_sum(ex, dst, num_segments=N_NODES)
    if per_head_slice:
        exw = jnp.repeat(ex, HD1, axis=1)
        payload = feat[src] * exw
    else:
        payload = (feat[src][:, None, :] * ex[:, :, None]).reshape(src.shape[0], H * D1)
    uagg = jax.ops.segment_sum(payload, dst, num_segments=N_NODES)
    return uagg, denom


def kernel(x, edge_index, W1, att_src1, att_dst1, b1, W2, att_src2, att_dst2, b2):
    loop = jnp.arange(N_NODES, dtype=edge_index.dtype)
    src = jnp.concatenate([edge_index[0], loop])
    dst = jnp.concatenate([edge_index[1], loop])

    xp1, a_src1, a_dst1, M1 = _dense1(
        x, W1, att_src1.reshape(1, D1), att_dst1.reshape(1, D1))
    uagg1, denom1 = _edge_phase(a_src1, a_dst1, M1, xp1, src, dst, True)
    hout, a_src2, a_dst2, M2 = _dense2(
        uagg1, denom1, b1.reshape(1, D1), W2,
        att_src2.reshape(1, H * HD2), att_dst2.reshape(1, H * HD2))
    uagg2, denom2 = _edge_phase(a_src2, a_dst2, M2, hout, src, dst, False)
    return _final(uagg2, denom2, W2, b2.reshape(1, HD2))
